# baseline (speedup 1.0000x reference)
import jax
import jax.numpy as jnp
from jax import lax
from jax.experimental import pallas as pl
from jax.experimental.pallas import tpu as pltpu
from jax.experimental.pallas import tpu_sc as plsc

D = 33
T = D * D * D
PT = 35968
H, W = 1080, 1920
N = H * W
NC, NS = 2, 16
NW = NC * NS
NPW = N // NW
C = 2400
CHUNKS = NPW // C


def _fuse_body(lc_ref, lut_ref, out_ref):
    lc = lc_ref[...]
    m = jnp.max(lc, axis=1, keepdims=True)
    e = jnp.exp(lc - m)
    w = e / jnp.sum(e, axis=1, keepdims=True)
    fused = jnp.sum(w[:, :, None] * lut_ref[...], axis=1)
    q = jnp.minimum((fused * 65535.0 + 0.5).astype(jnp.int32), 65535)
    b16 = jnp.right_shift(
        jax.lax.bitcast_convert_type(fused, jnp.int32) + 0x8000, 16)
    enc = jnp.concatenate([q[0:1], b16[1:2]], axis=0)
    hq = jnp.concatenate(
        [enc[:, 1:], jnp.zeros((2, 1), jnp.int32)], axis=1)
    word = jnp.bitwise_or(enc, jnp.left_shift(hq, 16))
    for c in range(3):
        out_ref[:, c, pl.ds(0, T)] = word[:, c * T:(c + 1) * T]


def _fuse_luts(lc, lut_r):
    return pl.pallas_call(
        _fuse_body,
        out_shape=jax.ShapeDtypeStruct((2, 3, PT), jnp.int32),
    )(lc, lut_r)


def _sc_pass_body(u16, img_hbm, tab_hbm, out_hbm,
                  tr, tg, tb, ir, ig, ib, orr, org, orb,
                  tab_sem, in_sem0, in_sem1, out_sem0, out_sem1):
    wid = lax.axis_index("s") * NC + lax.axis_index("c")
    base0 = wid * NPW

    td0 = pltpu.async_copy(tab_hbm.at[pl.ds(0, PT)], tr, tab_sem)
    td1 = pltpu.async_copy(tab_hbm.at[pl.ds(PT, PT)], tg, tab_sem)
    td2 = pltpu.async_copy(tab_hbm.at[pl.ds(2 * PT, PT)], tb, tab_sem)
    pltpu.async_copy(img_hbm.at[pl.ds(base0, C)], ir.at[pl.ds(0, C)], in_sem0)
    pltpu.async_copy(img_hbm.at[pl.ds(N + base0, C)], ig.at[pl.ds(0, C)],
                     in_sem0)
    pltpu.async_copy(img_hbm.at[pl.ds(2 * N + base0, C)], ib.at[pl.ds(0, C)],
                     in_sem0)
    td0.wait()
    td1.wait()
    td2.wait()

    def _wait_in(sem):
        for _ in range(3):
            pltpu.make_async_copy(img_hbm.at[pl.ds(0, C)],
                                  ir.at[pl.ds(0, C)], sem).wait()

    def _wait_out(sem):
        for _ in range(3):
            pltpu.make_async_copy(orr.at[pl.ds(0, C)],
                                  out_hbm.at[pl.ds(0, C)], sem).wait()

    def _compute(off):
        @plsc.parallel_loop(0, C // 16, unroll=2)
        def _px(i):
            sl = pl.ds(off + i * 16, 16)
            slo = pl.ds(i * 16, 16)
            r = ir[sl]
            g = ig[sl]
            b = ib[sl]
            xr = r * (D - 1.0)
            xg = g * (D - 1.0)
            xb = b * (D - 1.0)
            ri = jnp.minimum(xr.astype(jnp.int32), D - 2)
            gi = jnp.minimum(xg.astype(jnp.int32), D - 2)
            bi = jnp.minimum(xb.astype(jnp.int32), D - 2)
            fr = xr - ri.astype(jnp.float32)
            fg = xg - gi.astype(jnp.float32)
            fb = xb - bi.astype(jnp.float32)
            if u16:
                omr = (1.0 - fr) * (1.0 / 65535.0)
                frs = fr * (1.0 / 65535.0)
            else:
                omr = 1.0 - fr
                frs = fr
            omg = 1.0 - fg
            omb = 1.0 - fb
            w00 = omg * omb
            w01 = fg * omb
            w10 = omg * fb
            w11 = fg * fb
            lin = (bi * D + gi) * D + ri
            i00 = lin
            i01 = lin + D
            i10 = lin + D * D
            i11 = lin + (D * D + D)
            if u16:
                msk = jnp.int32(0xFFFF)

                def _pair(p):
                    lo = jnp.bitwise_and(p, msk).astype(jnp.float32)
                    hi = lax.shift_right_logical(p, 16).astype(jnp.float32)
                    return lo * omr + hi * frs
            else:
                hmsk = jnp.int32(-65536)

                def _pair(p):
                    lo = lax.bitcast_convert_type(
                        jnp.left_shift(p, 16), jnp.float32)
                    hi = lax.bitcast_convert_type(
                        jnp.bitwise_and(p, hmsk), jnp.float32)
                    return lo * omr + hi * frs

            outs = []
            for tab in (tr, tg, tb):
                a00 = _pair(plsc.load_gather(tab, [i00]))
                a01 = _pair(plsc.load_gather(tab, [i01]))
                a10 = _pair(plsc.load_gather(tab, [i10]))
                a11 = _pair(plsc.load_gather(tab, [i11]))
                outs.append((a00 * w00 + a01 * w01)
                            + (a10 * w10 + a11 * w11))
            orr[slo] = outs[0]
            org[slo] = outs[1]
            orb[slo] = outs[2]

    def _iter(t, off, my_in_sem, next_in_sem):
        noff = C - off

        @pl.when(t + 1 < CHUNKS)
        def _():
            nb = base0 + (t + 1) * C
            pltpu.async_copy(img_hbm.at[pl.ds(nb, C)],
                             ir.at[pl.ds(noff, C)], next_in_sem)
            pltpu.async_copy(img_hbm.at[pl.ds(N + nb, C)],
                             ig.at[pl.ds(noff, C)], next_in_sem)
            pltpu.async_copy(img_hbm.at[pl.ds(2 * N + nb, C)],
                             ib.at[pl.ds(noff, C)], next_in_sem)

        _wait_in(my_in_sem)

        @pl.when(t >= 1)
        def _():
            _wait_out(out_sem0)

        _compute(off)
        ob = base0 + t * C
        pltpu.async_copy(orr, out_hbm.at[pl.ds(ob, C)], out_sem0)
        pltpu.async_copy(org, out_hbm.at[pl.ds(N + ob, C)], out_sem0)
        pltpu.async_copy(orb, out_hbm.at[pl.ds(2 * N + ob, C)], out_sem0)

    @pl.loop(0, CHUNKS)
    def _chunk(t):
        slot = lax.rem(t, 2)

        @pl.when(slot == 0)
        def _():
            _iter(t, 0, in_sem0, in_sem1)

        @pl.when(slot == 1)
        def _():
            _iter(t, C, in_sem1, in_sem0)

    _wait_out(out_sem0)


def _make_sc_pass(u16):
    import functools
    return pl.kernel(
        functools.partial(_sc_pass_body, u16),
        out_type=jax.ShapeDtypeStruct((3 * N,), jnp.float32),
        mesh=plsc.VectorSubcoreMesh(
            core_axis_name="c", subcore_axis_name="s",
            num_cores=NC, num_subcores=NS),
        scratch_types=[
            pltpu.VMEM((PT,), jnp.int32),
            pltpu.VMEM((PT,), jnp.int32),
            pltpu.VMEM((PT,), jnp.int32),
            pltpu.VMEM((2 * C,), jnp.float32),
            pltpu.VMEM((2 * C,), jnp.float32),
            pltpu.VMEM((2 * C,), jnp.float32),
            pltpu.VMEM((C,), jnp.float32),
            pltpu.VMEM((C,), jnp.float32),
            pltpu.VMEM((C,), jnp.float32),
            pltpu.SemaphoreType.DMA,
            pltpu.SemaphoreType.DMA,
            pltpu.SemaphoreType.DMA,
            pltpu.SemaphoreType.DMA,
            pltpu.SemaphoreType.DMA,
        ],
        compiler_params=pltpu.CompilerParams(needs_layout_passes=False),
    )


_sc_pass_u16 = _make_sc_pass(True)
_sc_pass_b16 = _make_sc_pass(False)


def kernel(gt, lut, lc0, lc1):
    img = gt.reshape(3 * N)
    lut_r = lut.reshape(2, 8, 3 * T)
    lc = jnp.stack([lc0, lc1])
    cluts = _fuse_luts(lc, lut_r).reshape(2, 3 * PT)
    i_s = _sc_pass_u16(img, cluts[0])
    i_f = _sc_pass_b16(i_s, cluts[1])
    return i_f.reshape(3, H, W)


# speedup vs baseline: 1.0333x; 1.0333x over previous
import jax
import jax.numpy as jnp
from jax import lax
from jax.experimental import pallas as pl
from jax.experimental.pallas import tpu as pltpu
from jax.experimental.pallas import tpu_sc as plsc

D = 33
T = D * D * D
PT = 35968
H, W = 1080, 1920
N = H * W
NC, NS = 2, 16
NW = NC * NS
NPW = N // NW
C = 1440
CHUNKS = NPW // C


def _fuse_body(lc_ref, lut_ref, out_ref):
    lc = lc_ref[...]
    m = jnp.max(lc, axis=1, keepdims=True)
    e = jnp.exp(lc - m)
    w = e / jnp.sum(e, axis=1, keepdims=True)
    fused = jnp.sum(w[:, :, None] * lut_ref[...], axis=1)
    q = jnp.minimum((fused * 65535.0 + 0.5).astype(jnp.int32), 65535)
    b16 = jnp.right_shift(
        jax.lax.bitcast_convert_type(fused, jnp.int32) + 0x8000, 16)
    enc = jnp.concatenate([q[0:1], b16[1:2]], axis=0)
    hq = jnp.concatenate(
        [enc[:, 1:], jnp.zeros((2, 1), jnp.int32)], axis=1)
    word = jnp.bitwise_or(enc, jnp.left_shift(hq, 16))
    for c in range(3):
        out_ref[:, c, pl.ds(0, T)] = word[:, c * T:(c + 1) * T]


def _fuse_luts(lc, lut_r):
    return pl.pallas_call(
        _fuse_body,
        out_shape=jax.ShapeDtypeStruct((2, 3, PT), jnp.int32),
    )(lc, lut_r)


def _sc_pass_body(u16, img_hbm, tab_hbm, out_hbm,
                  tr, tg, tb, ir, ig, ib, orr, org, orb,
                  tab_sem, in_sem0, in_sem1, out_sem0, out_sem1):
    wid = lax.axis_index("s") * NC + lax.axis_index("c")
    base0 = wid * NPW

    td0 = pltpu.async_copy(tab_hbm.at[pl.ds(0, PT)], tr, tab_sem)
    td1 = pltpu.async_copy(tab_hbm.at[pl.ds(PT, PT)], tg, tab_sem)
    td2 = pltpu.async_copy(tab_hbm.at[pl.ds(2 * PT, PT)], tb, tab_sem)
    pltpu.async_copy(img_hbm.at[pl.ds(base0, C)], ir.at[pl.ds(0, C)], in_sem0)
    pltpu.async_copy(img_hbm.at[pl.ds(N + base0, C)], ig.at[pl.ds(0, C)],
                     in_sem0)
    pltpu.async_copy(img_hbm.at[pl.ds(2 * N + base0, C)], ib.at[pl.ds(0, C)],
                     in_sem0)
    td0.wait()
    td1.wait()
    td2.wait()

    def _wait_in(sem):
        for _ in range(3):
            pltpu.make_async_copy(img_hbm.at[pl.ds(0, C)],
                                  ir.at[pl.ds(0, C)], sem).wait()

    def _wait_out(sem):
        for _ in range(3):
            pltpu.make_async_copy(orr.at[pl.ds(0, C)],
                                  out_hbm.at[pl.ds(0, C)], sem).wait()

    def _compute(off):
        @plsc.parallel_loop(0, C // 16, unroll=2)
        def _px(i):
            sl = pl.ds(off + i * 16, 16)
            r = ir[sl]
            g = ig[sl]
            b = ib[sl]
            xr = r * (D - 1.0)
            xg = g * (D - 1.0)
            xb = b * (D - 1.0)
            ri = jnp.minimum(xr.astype(jnp.int32), D - 2)
            gi = jnp.minimum(xg.astype(jnp.int32), D - 2)
            bi = jnp.minimum(xb.astype(jnp.int32), D - 2)
            fr = xr - ri.astype(jnp.float32)
            fg = xg - gi.astype(jnp.float32)
            fb = xb - bi.astype(jnp.float32)
            if u16:
                omr = (1.0 - fr) * (1.0 / 65535.0)
                frs = fr * (1.0 / 65535.0)
            else:
                omr = 1.0 - fr
                frs = fr
            omg = 1.0 - fg
            omb = 1.0 - fb
            w00 = omg * omb
            w01 = fg * omb
            w10 = omg * fb
            w11 = fg * fb
            lin = (bi * D + gi) * D + ri
            i00 = lin
            i01 = lin + D
            i10 = lin + D * D
            i11 = lin + (D * D + D)
            if u16:
                msk = jnp.int32(0xFFFF)

                def _pair(p):
                    lo = jnp.bitwise_and(p, msk).astype(jnp.float32)
                    hi = lax.shift_right_logical(p, 16).astype(jnp.float32)
                    return lo * omr + hi * frs
            else:
                hmsk = jnp.int32(-65536)

                def _pair(p):
                    lo = lax.bitcast_convert_type(
                        jnp.left_shift(p, 16), jnp.float32)
                    hi = lax.bitcast_convert_type(
                        jnp.bitwise_and(p, hmsk), jnp.float32)
                    return lo * omr + hi * frs

            outs = []
            for tab in (tr, tg, tb):
                a00 = _pair(plsc.load_gather(tab, [i00]))
                a01 = _pair(plsc.load_gather(tab, [i01]))
                a10 = _pair(plsc.load_gather(tab, [i10]))
                a11 = _pair(plsc.load_gather(tab, [i11]))
                outs.append((a00 * w00 + a01 * w01)
                            + (a10 * w10 + a11 * w11))
            orr[sl] = outs[0]
            org[sl] = outs[1]
            orb[sl] = outs[2]

    def _iter(t, off, my_in_sem, next_in_sem, my_out_sem):
        noff = C - off

        @pl.when(t + 1 < CHUNKS)
        def _():
            nb = base0 + (t + 1) * C
            pltpu.async_copy(img_hbm.at[pl.ds(nb, C)],
                             ir.at[pl.ds(noff, C)], next_in_sem)
            pltpu.async_copy(img_hbm.at[pl.ds(N + nb, C)],
                             ig.at[pl.ds(noff, C)], next_in_sem)
            pltpu.async_copy(img_hbm.at[pl.ds(2 * N + nb, C)],
                             ib.at[pl.ds(noff, C)], next_in_sem)

        _wait_in(my_in_sem)

        @pl.when(t >= 2)
        def _():
            _wait_out(my_out_sem)

        _compute(off)
        ob = base0 + t * C
        pltpu.async_copy(orr.at[pl.ds(off, C)],
                         out_hbm.at[pl.ds(ob, C)], my_out_sem)
        pltpu.async_copy(org.at[pl.ds(off, C)],
                         out_hbm.at[pl.ds(N + ob, C)], my_out_sem)
        pltpu.async_copy(orb.at[pl.ds(off, C)],
                         out_hbm.at[pl.ds(2 * N + ob, C)], my_out_sem)

    @pl.loop(0, CHUNKS)
    def _chunk(t):
        slot = lax.rem(t, 2)

        @pl.when(slot == 0)
        def _():
            _iter(t, 0, in_sem0, in_sem1, out_sem0)

        @pl.when(slot == 1)
        def _():
            _iter(t, C, in_sem1, in_sem0, out_sem1)

    _wait_out(out_sem0)
    _wait_out(out_sem1)


def _make_sc_pass(u16):
    import functools
    return pl.kernel(
        functools.partial(_sc_pass_body, u16),
        out_type=jax.ShapeDtypeStruct((3 * N,), jnp.float32),
        mesh=plsc.VectorSubcoreMesh(
            core_axis_name="c", subcore_axis_name="s",
            num_cores=NC, num_subcores=NS),
        scratch_types=[
            pltpu.VMEM((PT,), jnp.int32),
            pltpu.VMEM((PT,), jnp.int32),
            pltpu.VMEM((PT,), jnp.int32),
            pltpu.VMEM((2 * C,), jnp.float32),
            pltpu.VMEM((2 * C,), jnp.float32),
            pltpu.VMEM((2 * C,), jnp.float32),
            pltpu.VMEM((2 * C,), jnp.float32),
            pltpu.VMEM((2 * C,), jnp.float32),
            pltpu.VMEM((2 * C,), jnp.float32),
            pltpu.SemaphoreType.DMA,
            pltpu.SemaphoreType.DMA,
            pltpu.SemaphoreType.DMA,
            pltpu.SemaphoreType.DMA,
            pltpu.SemaphoreType.DMA,
        ],
        compiler_params=pltpu.CompilerParams(needs_layout_passes=False),
    )


_sc_pass_u16 = _make_sc_pass(True)
_sc_pass_b16 = _make_sc_pass(False)


def kernel(gt, lut, lc0, lc1):
    img = gt.reshape(3 * N)
    lut_r = lut.reshape(2, 8, 3 * T)
    lc = jnp.stack([lc0, lc1])
    cluts = _fuse_luts(lc, lut_r).reshape(2, 3 * PT)
    i_s = _sc_pass_u16(img, cluts[0])
    i_f = _sc_pass_b16(i_s, cluts[1])
    return i_f.reshape(3, H, W)


# speedup vs baseline: 1.0700x; 1.0356x over previous
import jax
import jax.numpy as jnp
from jax import lax
from jax.experimental import pallas as pl
from jax.experimental.pallas import tpu as pltpu
from jax.experimental.pallas import tpu_sc as plsc

D = 33
T = D * D * D
PT = 35968
H, W = 1080, 1920
N = H * W
NC, NS = 2, 16
NW = NC * NS
NPW = N // NW
C = 1440
CHUNKS = NPW // C


def _fuse_body(lc_ref, lut_ref, out_ref):
    lc = lc_ref[...]
    m = jnp.max(lc, axis=1, keepdims=True)
    e = jnp.exp(lc - m)
    w = e / jnp.sum(e, axis=1, keepdims=True)
    lut = lut_ref[...]
    fused6 = jnp.sum(w[:, :, None, None, None, None] * lut, axis=1)
    fused = fused6.reshape(2, 3 * T)
    q = jnp.minimum((fused * 65535.0 + 0.5).astype(jnp.int32), 65535)
    b16 = jnp.right_shift(
        jax.lax.bitcast_convert_type(fused, jnp.int32) + 0x8000, 16)
    enc = jnp.concatenate([q[0:1], b16[1:2]], axis=0)
    hq = jnp.concatenate(
        [enc[:, 1:], jnp.zeros((2, 1), jnp.int32)], axis=1)
    word = jnp.bitwise_or(enc, jnp.left_shift(hq, 16))
    for c in range(3):
        out_ref[:, c, pl.ds(0, T)] = word[:, c * T:(c + 1) * T]


def _fuse_luts(lc, lut_r):
    return pl.pallas_call(
        _fuse_body,
        out_shape=jax.ShapeDtypeStruct((2, 3, PT), jnp.int32),
    )(lc, lut_r)


def _sc_pass_body(u16, img_hbm, tab_hbm, out_hbm,
                  tr, tg, tb, ir, ig, ib, orr, org, orb,
                  tab_sem, in_sem0, in_sem1, out_sem0, out_sem1):
    wid = lax.axis_index("s") * NC + lax.axis_index("c")
    base0 = wid * NPW

    td0 = pltpu.async_copy(tab_hbm.at[pl.ds(0, PT)], tr, tab_sem)
    td1 = pltpu.async_copy(tab_hbm.at[pl.ds(PT, PT)], tg, tab_sem)
    td2 = pltpu.async_copy(tab_hbm.at[pl.ds(2 * PT, PT)], tb, tab_sem)
    pltpu.async_copy(img_hbm.at[pl.ds(base0, C)], ir.at[pl.ds(0, C)], in_sem0)
    pltpu.async_copy(img_hbm.at[pl.ds(N + base0, C)], ig.at[pl.ds(0, C)],
                     in_sem0)
    pltpu.async_copy(img_hbm.at[pl.ds(2 * N + base0, C)], ib.at[pl.ds(0, C)],
                     in_sem0)
    td0.wait()
    td1.wait()
    td2.wait()

    def _wait_in(sem):
        for _ in range(3):
            pltpu.make_async_copy(img_hbm.at[pl.ds(0, C)],
                                  ir.at[pl.ds(0, C)], sem).wait()

    def _wait_out(sem):
        for _ in range(3):
            pltpu.make_async_copy(orr.at[pl.ds(0, C)],
                                  out_hbm.at[pl.ds(0, C)], sem).wait()

    def _compute(off):
        @plsc.parallel_loop(0, C // 16, unroll=2)
        def _px(i):
            sl = pl.ds(off + i * 16, 16)
            r = ir[sl]
            g = ig[sl]
            b = ib[sl]
            xr = r * (D - 1.0)
            xg = g * (D - 1.0)
            xb = b * (D - 1.0)
            ri = jnp.minimum(xr.astype(jnp.int32), D - 2)
            gi = jnp.minimum(xg.astype(jnp.int32), D - 2)
            bi = jnp.minimum(xb.astype(jnp.int32), D - 2)
            fr = xr - ri.astype(jnp.float32)
            fg = xg - gi.astype(jnp.float32)
            fb = xb - bi.astype(jnp.float32)
            if u16:
                omr = (1.0 - fr) * (1.0 / 65535.0)
                frs = fr * (1.0 / 65535.0)
            else:
                omr = 1.0 - fr
                frs = fr
            omg = 1.0 - fg
            omb = 1.0 - fb
            w00 = omg * omb
            w01 = fg * omb
            w10 = omg * fb
            w11 = fg * fb
            lin = (bi * D + gi) * D + ri
            i00 = lin
            i01 = lin + D
            i10 = lin + D * D
            i11 = lin + (D * D + D)
            if u16:
                msk = jnp.int32(0xFFFF)

                def _pair(p):
                    lo = jnp.bitwise_and(p, msk).astype(jnp.float32)
                    hi = lax.shift_right_logical(p, 16).astype(jnp.float32)
                    return lo * omr + hi * frs
            else:
                hmsk = jnp.int32(-65536)

                def _pair(p):
                    lo = lax.bitcast_convert_type(
                        jnp.left_shift(p, 16), jnp.float32)
                    hi = lax.bitcast_convert_type(
                        jnp.bitwise_and(p, hmsk), jnp.float32)
                    return lo * omr + hi * frs

            outs = []
            for tab in (tr, tg, tb):
                a00 = _pair(plsc.load_gather(tab, [i00]))
                a01 = _pair(plsc.load_gather(tab, [i01]))
                a10 = _pair(plsc.load_gather(tab, [i10]))
                a11 = _pair(plsc.load_gather(tab, [i11]))
                outs.append((a00 * w00 + a01 * w01)
                            + (a10 * w10 + a11 * w11))
            orr[sl] = outs[0]
            org[sl] = outs[1]
            orb[sl] = outs[2]

    def _iter(t, off, my_in_sem, next_in_sem, my_out_sem):
        noff = C - off

        @pl.when(t + 1 < CHUNKS)
        def _():
            nb = base0 + (t + 1) * C
            pltpu.async_copy(img_hbm.at[pl.ds(nb, C)],
                             ir.at[pl.ds(noff, C)], next_in_sem)
            pltpu.async_copy(img_hbm.at[pl.ds(N + nb, C)],
                             ig.at[pl.ds(noff, C)], next_in_sem)
            pltpu.async_copy(img_hbm.at[pl.ds(2 * N + nb, C)],
                             ib.at[pl.ds(noff, C)], next_in_sem)

        _wait_in(my_in_sem)

        @pl.when(t >= 2)
        def _():
            _wait_out(my_out_sem)

        _compute(off)
        ob = base0 + t * C
        pltpu.async_copy(orr.at[pl.ds(off, C)],
                         out_hbm.at[pl.ds(ob, C)], my_out_sem)
        pltpu.async_copy(org.at[pl.ds(off, C)],
                         out_hbm.at[pl.ds(N + ob, C)], my_out_sem)
        pltpu.async_copy(orb.at[pl.ds(off, C)],
                         out_hbm.at[pl.ds(2 * N + ob, C)], my_out_sem)

    @pl.loop(0, CHUNKS)
    def _chunk(t):
        slot = lax.rem(t, 2)

        @pl.when(slot == 0)
        def _():
            _iter(t, 0, in_sem0, in_sem1, out_sem0)

        @pl.when(slot == 1)
        def _():
            _iter(t, C, in_sem1, in_sem0, out_sem1)

    _wait_out(out_sem0)
    _wait_out(out_sem1)


def _make_sc_pass(u16):
    import functools
    return pl.kernel(
        functools.partial(_sc_pass_body, u16),
        out_type=jax.ShapeDtypeStruct((3 * N,), jnp.float32),
        mesh=plsc.VectorSubcoreMesh(
            core_axis_name="c", subcore_axis_name="s",
            num_cores=NC, num_subcores=NS),
        scratch_types=[
            pltpu.VMEM((PT,), jnp.int32),
            pltpu.VMEM((PT,), jnp.int32),
            pltpu.VMEM((PT,), jnp.int32),
            pltpu.VMEM((2 * C,), jnp.float32),
            pltpu.VMEM((2 * C,), jnp.float32),
            pltpu.VMEM((2 * C,), jnp.float32),
            pltpu.VMEM((2 * C,), jnp.float32),
            pltpu.VMEM((2 * C,), jnp.float32),
            pltpu.VMEM((2 * C,), jnp.float32),
            pltpu.SemaphoreType.DMA,
            pltpu.SemaphoreType.DMA,
            pltpu.SemaphoreType.DMA,
            pltpu.SemaphoreType.DMA,
            pltpu.SemaphoreType.DMA,
        ],
        compiler_params=pltpu.CompilerParams(needs_layout_passes=False),
    )


_sc_pass_u16 = _make_sc_pass(True)
_sc_pass_b16 = _make_sc_pass(False)


def kernel(gt, lut, lc0, lc1):
    img = gt.reshape(3 * N)
    lc = jnp.stack([lc0, lc1])
    cluts = _fuse_luts(lc, lut).reshape(2, 3 * PT)
    i_s = _sc_pass_u16(img, cluts[0])
    i_f = _sc_pass_b16(i_s, cluts[1])
    return i_f.reshape(3, H, W)
